# baseline (speedup 1.0000x reference)
import functools
import math

import jax
import jax.numpy as jnp
from jax import lax
from jax.experimental import pallas as pl
from jax.experimental.pallas import tpu as pltpu
from jax.experimental.pallas import tpu_sc as plsc

_INV_SQRT2 = 1.0 / math.sqrt(2.0)


def _gelu(x):
    return 0.5 * x * (1.0 + lax.erf(x * _INV_SQRT2))


def _ln_rows(x, g, b, eps=1e-5):
    mu = jnp.mean(x, axis=1, keepdims=True)
    var = jnp.mean(jnp.square(x - mu), axis=1, keepdims=True)
    return (x - mu) * lax.rsqrt(var + eps) * g + b


def _dot_t(a, b):
    return lax.dot_general(a, b, (((1,), (1,)), ((), ())),
                           preferred_element_type=jnp.float32)


def _router_body(x_ref, wr1_ref, wr2_ref, win_ref, g_ref, b_ref,
                 sel_ref, acts_ref):
    xv = x_ref[...]
    xn = _ln_rows(xv, g_ref[...], b_ref[...])
    h = _gelu(_dot_t(xn, wr1_ref[...]))
    scores = _dot_t(h, wr2_ref[...])
    pre = _dot_t(xv.astype(jnp.bfloat16), win_ref[...])

    tsz, dff = scores.shape
    lane = lax.broadcasted_iota(jnp.int32, (tsz, dff), 1)
    col8 = lax.broadcasted_iota(jnp.int32, (tsz, 8), 1)
    sel_acc = jnp.zeros((tsz, 8), jnp.int32)
    act_acc = jnp.zeros((tsz, 8), jnp.float32)
    neg_inf = jnp.float32(-jnp.inf)
    for j in range(8):
        m = jnp.max(scores, axis=1, keepdims=True)
        is_m = scores == m
        idx = jnp.min(jnp.where(is_m, lane, dff), axis=1, keepdims=True)
        aj = jnp.sum(jnp.where(is_m, pre, 0.0), axis=1, keepdims=True)
        scores = jnp.where(is_m, neg_inf, scores)
        sel_acc = jnp.where(col8 == j, idx, sel_acc)
        act_acc = jnp.where(col8 == j, aj, act_acc)
    sel_ref[...] = sel_acc
    acts_ref[...] = _gelu(act_acc)


def _router_topk(x_flat, Wr1, Wr2, W_in, ln_g, ln_b, t_blk=128):
    n, d_model = x_flat.shape
    d_ff = W_in.shape[0]
    grid = (n // t_blk,)
    return pl.pallas_call(
        _router_body,
        grid=grid,
        in_specs=[
            pl.BlockSpec((t_blk, d_model), lambda t: (t, 0)),
            pl.BlockSpec((d_model, d_model), lambda t: (0, 0)),
            pl.BlockSpec((d_ff, d_model), lambda t: (0, 0)),
            pl.BlockSpec((d_ff, d_model), lambda t: (0, 0)),
            pl.BlockSpec((1, d_model), lambda t: (0, 0)),
            pl.BlockSpec((1, d_model), lambda t: (0, 0)),
        ],
        out_specs=[
            pl.BlockSpec((t_blk, 8), lambda t: (t, 0)),
            pl.BlockSpec((t_blk, 8), lambda t: (t, 0)),
        ],
        out_shape=[
            jax.ShapeDtypeStruct((n, 8), jnp.int32),
            jax.ShapeDtypeStruct((n, 8), jnp.float32),
        ],
        compiler_params=pltpu.CompilerParams(
            dimension_semantics=("arbitrary",)),
    )(x_flat, Wr1, Wr2, W_in.astype(jnp.bfloat16), ln_g.reshape(1, -1),
      ln_b.reshape(1, -1))


def _sc_gather(table, idx):
    v, d = table.shape
    b = idx.shape[0]
    info = plsc.get_sparse_core_info()
    nw = info.num_cores * info.num_subcores
    b_per_w = b // nw
    chunk = 256
    n_chunks = b_per_w // chunk
    mesh = plsc.VectorSubcoreMesh(core_axis_name="c", subcore_axis_name="s")

    @functools.partial(
        pl.kernel,
        mesh=mesh,
        out_type=jax.ShapeDtypeStruct((b, d), jnp.float32),
        scratch_types=[
            pltpu.VMEM((b_per_w,), jnp.int32),
            pltpu.VMEM((chunk, d), jnp.float32),
            pltpu.VMEM((chunk, d), jnp.float32),
            pltpu.SemaphoreType.DMA,
            pltpu.SemaphoreType.DMA,
        ],
    )
    def gather_kernel(table_hbm, idx_hbm, out_hbm, idx_v, rows0, rows1, sem0,
                      sem1):
        wid = lax.axis_index("s") * info.num_cores + lax.axis_index("c")
        base = wid * b_per_w
        pltpu.sync_copy(idx_hbm.at[pl.ds(base, b_per_w)], idx_v)
        bufs = (rows0, rows1)
        sems = (sem0, sem1)
        copies = [pltpu.async_copy(
            table_hbm.at[idx_v.at[pl.ds(0, chunk)]], bufs[0], sems[0])]
        for c in range(n_chunks):
            if c + 1 < n_chunks:
                copies.append(pltpu.async_copy(
                    table_hbm.at[idx_v.at[pl.ds((c + 1) * chunk, chunk)]],
                    bufs[(c + 1) % 2], sems[(c + 1) % 2]))
            copies[c].wait()
            pltpu.sync_copy(bufs[c % 2],
                            out_hbm.at[pl.ds(base + c * chunk, chunk)])

    return gather_kernel(table, idx)


def _phi_rho_body(nv_ref, act_ref, w1a_ref, wact_ref, b1_ref, g1_ref, bb1_ref,
                  w2_ref, b2_ref, g2_ref, bb2_ref, rg_ref, rb_ref,
                  rw1_ref, rb1_ref, rw2_ref, rb2_ref, out_ref):
    t1 = _dot_t(nv_ref[...], w1a_ref[...])
    t1 = t1 + act_ref[...] * wact_ref[...] + b1_ref[...]
    t1 = _gelu(_ln_rows(t1, g1_ref[...], bb1_ref[...]))
    t2 = _dot_t(t1, w2_ref[...]) + b2_ref[...]
    t2 = _ln_rows(t2, g2_ref[...], bb2_ref[...])
    rows = t2.shape[0]
    agg = jnp.sum(t2.reshape(rows // 8, 8, t2.shape[1]), axis=1)
    r = _ln_rows(agg, rg_ref[...], rb_ref[...])
    r = _gelu(_dot_t(r, rw1_ref[...]) + rb1_ref[...])
    out_ref[...] = _dot_t(r, rw2_ref[...]) + rb2_ref[...]


def _phi_rho(nv_sel, acts_col, phi_W1a, phi_wact, phi_b1, phi_ln1_g, phi_ln1_b,
             phi_W2, phi_b2, phi_ln2_g, phi_ln2_b, rho_ln_g, rho_ln_b,
             rho_W1, rho_b1, rho_W2, rho_b2, t_blk=512):
    rows, d_n = nv_sel.shape
    n = rows // 8
    d_h = phi_W2.shape[0]
    d_h2 = rho_W1.shape[0]
    d_model = rho_W2.shape[0]
    grid = (n // t_blk,)
    row_blk = t_blk * 8

    def full(shape):
        return pl.BlockSpec(shape, lambda t: tuple(0 for _ in shape))

    return pl.pallas_call(
        _phi_rho_body,
        grid=grid,
        in_specs=[
            pl.BlockSpec((row_blk, d_n), lambda t: (t, 0)),
            pl.BlockSpec((row_blk, 1), lambda t: (t, 0)),
            full((d_h, d_n)),
            full((1, d_h)),
            full((1, d_h)),
            full((1, d_h)),
            full((1, d_h)),
            full((d_h, d_h)),
            full((1, d_h)),
            full((1, d_h)),
            full((1, d_h)),
            full((1, d_h)),
            full((1, d_h)),
            full((d_h2, d_h)),
            full((1, d_h2)),
            full((d_model, d_h2)),
            full((1, d_model)),
        ],
        out_specs=pl.BlockSpec((t_blk, d_model), lambda t: (t, 0)),
        out_shape=jax.ShapeDtypeStruct((n, d_model), jnp.float32),
    )(nv_sel, acts_col, phi_W1a, phi_wact.reshape(1, -1),
      phi_b1.reshape(1, -1), phi_ln1_g.reshape(1, -1),
      phi_ln1_b.reshape(1, -1), phi_W2, phi_b2.reshape(1, -1),
      phi_ln2_g.reshape(1, -1), phi_ln2_b.reshape(1, -1),
      rho_ln_g.reshape(1, -1), rho_ln_b.reshape(1, -1),
      rho_W1, rho_b1.reshape(1, -1), rho_W2, rho_b2.reshape(1, -1))


def kernel(x, neuron_vecs, W_in, Wr1, Wr2, ln_r_g, ln_r_b, phi_W1, phi_b1,
           phi_ln1_g, phi_ln1_b, phi_W2, phi_b2, phi_ln2_g, phi_ln2_b,
           rho_ln_g, rho_ln_b, rho_W1, rho_b1, rho_W2, rho_b2, top_k):
    batch, seq, d_model = x.shape
    n = batch * seq
    x_flat = x.reshape(n, d_model)

    sel, acts = _router_topk(x_flat, Wr1, Wr2, W_in, ln_r_g, ln_r_b)

    nv_sel = _sc_gather(neuron_vecs, sel.reshape(n * 8))

    out = _phi_rho(
        nv_sel, acts.reshape(n * 8, 1),
        phi_W1[:, :neuron_vecs.shape[1]], phi_W1[:, neuron_vecs.shape[1]],
        phi_b1, phi_ln1_g, phi_ln1_b, phi_W2, phi_b2, phi_ln2_g, phi_ln2_b,
        rho_ln_g, rho_ln_b, rho_W1, rho_b1, rho_W2, rho_b2)
    return out.reshape(batch, seq, d_model)


# speedup vs baseline: 1.0917x; 1.0917x over previous
import functools
import math

import jax
import jax.numpy as jnp
from jax import lax
from jax.experimental import pallas as pl
from jax.experimental.pallas import tpu as pltpu
from jax.experimental.pallas import tpu_sc as plsc

_INV_SQRT2 = 1.0 / math.sqrt(2.0)


def _gelu(x):
    return 0.5 * x * (1.0 + lax.erf(x * _INV_SQRT2))


def _ln_rows(x, g, b, eps=1e-5):
    mu = jnp.mean(x, axis=1, keepdims=True)
    var = jnp.mean(jnp.square(x - mu), axis=1, keepdims=True)
    return (x - mu) * lax.rsqrt(var + eps) * g + b


def _dot_t(a, b):
    return lax.dot_general(a, b, (((1,), (1,)), ((), ())),
                           preferred_element_type=jnp.float32)


def _router_body(x_ref, wr1_ref, wr2_ref, win_ref, g_ref, b_ref,
                 sel_ref, acts_ref, h_s, xb_s, sc_s, pa_s, sel_s, act_s):
    t = pl.program_id(0)
    f = pl.program_id(1)
    nt = pl.num_programs(0)
    nf = pl.num_programs(1)
    fsz = wr2_ref.shape[0]
    cur = lax.rem(t, 2)
    prv = lax.rem(t + 1, 2)

    @pl.when(t < nt - 1)
    def _():
        @pl.when(f == 0)
        def _():
            xv = x_ref[...]
            xn = _ln_rows(xv, g_ref[...], b_ref[...])
            h_s[...] = _gelu(_dot_t(xn, wr1_ref[...]))
            xb_s[...] = xv.astype(jnp.bfloat16)
        sc_s[cur, :, pl.ds(f * fsz, fsz)] = _dot_t(h_s[...], wr2_ref[...])
        pa_s[cur, :, pl.ds(f * fsz, fsz)] = _dot_t(
            xb_s[...], win_ref[...]).astype(jnp.bfloat16)

    @pl.when(t > 0)
    def _():
        scores = sc_s[prv]
        pre = pa_s[prv]
        tsz, dff = scores.shape
        lane = lax.broadcasted_iota(jnp.int32, (tsz, dff), 1)
        col8 = lax.broadcasted_iota(jnp.int32, (tsz, 8), 1)
        m = jnp.max(scores, axis=1, keepdims=True)
        is_m = scores == m
        idx = jnp.min(jnp.where(is_m, lane, dff), axis=1, keepdims=True)
        aj = jnp.sum(jnp.where(is_m, pre, jnp.bfloat16(0.0)), axis=1,
                     keepdims=True).astype(jnp.float32)
        sc_s[prv] = jnp.where(is_m, jnp.float32(-jnp.inf), scores)
        sel_prev = jnp.where(f == 0, jnp.zeros((tsz, 8), jnp.int32),
                             sel_s[prv])
        act_prev = jnp.where(f == 0, jnp.zeros((tsz, 8), jnp.float32),
                             act_s[prv])
        sel_s[prv] = jnp.where(col8 == f, idx, sel_prev)
        act_s[prv] = jnp.where(col8 == f, aj, act_prev)

        @pl.when(f == nf - 1)
        def _():
            sel_ref[...] = sel_s[prv]
            acts_ref[...] = _gelu(act_s[prv])


def _router_topk(x_flat, Wr1, Wr2, W_in, ln_g, ln_b, t_blk=256, f_blk=512):
    n, d_model = x_flat.shape
    d_ff = W_in.shape[0]
    nt = n // t_blk
    grid = (nt + 1, d_ff // f_blk)
    last = nt - 1
    return pl.pallas_call(
        _router_body,
        grid=grid,
        in_specs=[
            pl.BlockSpec((t_blk, d_model),
                         lambda t, f: (jnp.minimum(t, last), 0)),
            pl.BlockSpec((d_model, d_model), lambda t, f: (0, 0)),
            pl.BlockSpec((f_blk, d_model), lambda t, f: (f, 0)),
            pl.BlockSpec((f_blk, d_model), lambda t, f: (f, 0)),
            pl.BlockSpec((1, d_model), lambda t, f: (0, 0)),
            pl.BlockSpec((1, d_model), lambda t, f: (0, 0)),
        ],
        out_specs=[
            pl.BlockSpec((t_blk, 8), lambda t, f: (jnp.maximum(t - 1, 0), 0)),
            pl.BlockSpec((t_blk, 8), lambda t, f: (jnp.maximum(t - 1, 0), 0)),
        ],
        out_shape=[
            jax.ShapeDtypeStruct((n, 8), jnp.int32),
            jax.ShapeDtypeStruct((n, 8), jnp.float32),
        ],
        scratch_shapes=[
            pltpu.VMEM((t_blk, d_model), jnp.float32),
            pltpu.VMEM((t_blk, d_model), jnp.bfloat16),
            pltpu.VMEM((2, t_blk, d_ff), jnp.float32),
            pltpu.VMEM((2, t_blk, d_ff), jnp.bfloat16),
            pltpu.VMEM((2, t_blk, 8), jnp.int32),
            pltpu.VMEM((2, t_blk, 8), jnp.float32),
        ],
        compiler_params=pltpu.CompilerParams(
            dimension_semantics=("arbitrary", "arbitrary")),
    )(x_flat, Wr1, Wr2, W_in.astype(jnp.bfloat16), ln_g.reshape(1, -1),
      ln_b.reshape(1, -1))


def _sc_gather(table, idx):
    v, d = table.shape
    b = idx.shape[0]
    info = plsc.get_sparse_core_info()
    nw = info.num_cores * info.num_subcores
    b_per_w = b // nw
    chunk = 256
    n_chunks = b_per_w // chunk
    mesh = plsc.VectorSubcoreMesh(core_axis_name="c", subcore_axis_name="s")

    @functools.partial(
        pl.kernel,
        mesh=mesh,
        out_type=jax.ShapeDtypeStruct((b, d), jnp.float32),
        scratch_types=[
            pltpu.VMEM((b_per_w,), jnp.int32),
            pltpu.VMEM((chunk, d), jnp.float32),
            pltpu.VMEM((chunk, d), jnp.float32),
            pltpu.SemaphoreType.DMA,
            pltpu.SemaphoreType.DMA,
        ],
    )
    def gather_kernel(table_hbm, idx_hbm, out_hbm, idx_v, rows0, rows1, sem0,
                      sem1):
        wid = lax.axis_index("s") * info.num_cores + lax.axis_index("c")
        base = wid * b_per_w
        pltpu.sync_copy(idx_hbm.at[pl.ds(base, b_per_w)], idx_v)
        bufs = (rows0, rows1)
        sems = (sem0, sem1)
        copies = [pltpu.async_copy(
            table_hbm.at[idx_v.at[pl.ds(0, chunk)]], bufs[0], sems[0])]
        for c in range(n_chunks):
            if c + 1 < n_chunks:
                copies.append(pltpu.async_copy(
                    table_hbm.at[idx_v.at[pl.ds((c + 1) * chunk, chunk)]],
                    bufs[(c + 1) % 2], sems[(c + 1) % 2]))
            copies[c].wait()
            pltpu.sync_copy(bufs[c % 2],
                            out_hbm.at[pl.ds(base + c * chunk, chunk)])

    return gather_kernel(table, idx)


def _phi_rho_body(nv_ref, act_ref, w1a_ref, wact_ref, b1_ref, g1_ref, bb1_ref,
                  w2_ref, b2_ref, g2_ref, bb2_ref, rg_ref, rb_ref,
                  rw1_ref, rb1_ref, rw2_ref, rb2_ref, out_ref):
    t1 = _dot_t(nv_ref[...], w1a_ref[...])
    t1 = t1 + act_ref[...] * wact_ref[...] + b1_ref[...]
    t1 = _gelu(_ln_rows(t1, g1_ref[...], bb1_ref[...]))
    t2 = _dot_t(t1, w2_ref[...]) + b2_ref[...]
    t2 = _ln_rows(t2, g2_ref[...], bb2_ref[...])
    rows = t2.shape[0]
    agg = jnp.sum(t2.reshape(rows // 8, 8, t2.shape[1]), axis=1)
    r = _ln_rows(agg, rg_ref[...], rb_ref[...])
    r = _gelu(_dot_t(r, rw1_ref[...]) + rb1_ref[...])
    out_ref[...] = _dot_t(r, rw2_ref[...]) + rb2_ref[...]


def _phi_rho(nv_sel, acts_col, phi_W1a, phi_wact, phi_b1, phi_ln1_g, phi_ln1_b,
             phi_W2, phi_b2, phi_ln2_g, phi_ln2_b, rho_ln_g, rho_ln_b,
             rho_W1, rho_b1, rho_W2, rho_b2, t_blk=512):
    rows, d_n = nv_sel.shape
    n = rows // 8
    d_h = phi_W2.shape[0]
    d_h2 = rho_W1.shape[0]
    d_model = rho_W2.shape[0]
    grid = (n // t_blk,)
    row_blk = t_blk * 8

    def full(shape):
        return pl.BlockSpec(shape, lambda t: tuple(0 for _ in shape))

    return pl.pallas_call(
        _phi_rho_body,
        grid=grid,
        in_specs=[
            pl.BlockSpec((row_blk, d_n), lambda t: (t, 0)),
            pl.BlockSpec((row_blk, 1), lambda t: (t, 0)),
            full((d_h, d_n)),
            full((1, d_h)),
            full((1, d_h)),
            full((1, d_h)),
            full((1, d_h)),
            full((d_h, d_h)),
            full((1, d_h)),
            full((1, d_h)),
            full((1, d_h)),
            full((1, d_h)),
            full((1, d_h)),
            full((d_h2, d_h)),
            full((1, d_h2)),
            full((d_model, d_h2)),
            full((1, d_model)),
        ],
        out_specs=pl.BlockSpec((t_blk, d_model), lambda t: (t, 0)),
        out_shape=jax.ShapeDtypeStruct((n, d_model), jnp.float32),
    )(nv_sel, acts_col, phi_W1a, phi_wact.reshape(1, -1),
      phi_b1.reshape(1, -1), phi_ln1_g.reshape(1, -1),
      phi_ln1_b.reshape(1, -1), phi_W2, phi_b2.reshape(1, -1),
      phi_ln2_g.reshape(1, -1), phi_ln2_b.reshape(1, -1),
      rho_ln_g.reshape(1, -1), rho_ln_b.reshape(1, -1),
      rho_W1, rho_b1.reshape(1, -1), rho_W2, rho_b2.reshape(1, -1))


def kernel(x, neuron_vecs, W_in, Wr1, Wr2, ln_r_g, ln_r_b, phi_W1, phi_b1,
           phi_ln1_g, phi_ln1_b, phi_W2, phi_b2, phi_ln2_g, phi_ln2_b,
           rho_ln_g, rho_ln_b, rho_W1, rho_b1, rho_W2, rho_b2, top_k):
    batch, seq, d_model = x.shape
    n = batch * seq
    x_flat = x.reshape(n, d_model)

    sel, acts = _router_topk(x_flat, Wr1, Wr2, W_in, ln_r_g, ln_r_b)

    nv_sel = _sc_gather(neuron_vecs, sel.reshape(n * 8))

    out = _phi_rho(
        nv_sel, acts.reshape(n * 8, 1),
        phi_W1[:, :neuron_vecs.shape[1]], phi_W1[:, neuron_vecs.shape[1]],
        phi_b1, phi_ln1_g, phi_ln1_b, phi_W2, phi_b2, phi_ln2_g, phi_ln2_b,
        rho_ln_g, rho_ln_b, rho_W1, rho_b1, rho_W2, rho_b2)
    return out.reshape(batch, seq, d_model)


# speedup vs baseline: 1.1165x; 1.0227x over previous
import functools
import math

import jax
import jax.numpy as jnp
from jax import lax
from jax.experimental import pallas as pl
from jax.experimental.pallas import tpu as pltpu
from jax.experimental.pallas import tpu_sc as plsc

_INV_SQRT2 = 1.0 / math.sqrt(2.0)


def _gelu(x):
    return 0.5 * x * (1.0 + lax.erf(x * _INV_SQRT2))


def _ln_rows(x, g, b, eps=1e-5):
    mu = jnp.mean(x, axis=1, keepdims=True)
    var = jnp.mean(jnp.square(x - mu), axis=1, keepdims=True)
    return (x - mu) * lax.rsqrt(var + eps) * g + b


def _dot_t(a, b):
    return lax.dot_general(a, b, (((1,), (1,)), ((), ())),
                           preferred_element_type=jnp.float32)


def _router_body(x_ref, wr1_ref, wr2_ref, win_ref, g_ref, b_ref,
                 sel_ref, acts_ref, h_s, xb_s, sc_s, pa_s, sel_s, act_s):
    t = pl.program_id(0)
    f = pl.program_id(1)
    nt = pl.num_programs(0)
    nf = pl.num_programs(1)
    fsz = wr2_ref.shape[0]
    cur = lax.rem(t, 2)
    prv = lax.rem(t + 1, 2)

    @pl.when(f == 0)
    def _():
        xv = x_ref[...]
        xn = _ln_rows(xv, g_ref[...], b_ref[...])
        h_s[...] = _gelu(_dot_t(xn, wr1_ref[...]))
        xb_s[...] = xv.astype(jnp.bfloat16)

    sc_s[cur, :, pl.ds(f * fsz, fsz)] = _dot_t(h_s[...], wr2_ref[...])
    pa_s[cur, :, pl.ds(f * fsz, fsz)] = _dot_t(
        xb_s[...], win_ref[...]).astype(jnp.bfloat16)

    scores = sc_s[prv]
    pre = pa_s[prv]
    tsz, dff = scores.shape
    lane = lax.broadcasted_iota(jnp.int32, (tsz, dff), 1)
    col8 = lax.broadcasted_iota(jnp.int32, (tsz, 8), 1)
    m = jnp.max(scores, axis=1, keepdims=True)
    is_m = scores == m
    idx = jnp.min(jnp.where(is_m, lane, dff), axis=1, keepdims=True)
    aj = jnp.sum(jnp.where(is_m, pre, jnp.bfloat16(0.0)), axis=1,
                 keepdims=True).astype(jnp.float32)
    sc_s[prv] = jnp.where(is_m, jnp.float32(-jnp.inf), scores)
    sel_prev = jnp.where(f == 0, jnp.zeros((tsz, 8), jnp.int32), sel_s[prv])
    act_prev = jnp.where(f == 0, jnp.zeros((tsz, 8), jnp.float32), act_s[prv])
    sel_s[prv] = jnp.where(col8 == f, idx, sel_prev)
    act_s[prv] = jnp.where(col8 == f, aj, act_prev)

    @pl.when(f == nf - 1)
    def _():
        sel_ref[...] = sel_s[prv]
        acts_ref[...] = _gelu(act_s[prv])


def _router_topk(x_flat, Wr1, Wr2, W_in, ln_g, ln_b, t_blk=256, f_blk=512):
    n, d_model = x_flat.shape
    d_ff = W_in.shape[0]
    nt = n // t_blk
    grid = (nt + 1, d_ff // f_blk)
    last = nt - 1
    return pl.pallas_call(
        _router_body,
        grid=grid,
        in_specs=[
            pl.BlockSpec((t_blk, d_model),
                         lambda t, f: (jnp.minimum(t, last), 0)),
            pl.BlockSpec((d_model, d_model), lambda t, f: (0, 0)),
            pl.BlockSpec((f_blk, d_model), lambda t, f: (f, 0)),
            pl.BlockSpec((f_blk, d_model), lambda t, f: (f, 0)),
            pl.BlockSpec((1, d_model), lambda t, f: (0, 0)),
            pl.BlockSpec((1, d_model), lambda t, f: (0, 0)),
        ],
        out_specs=[
            pl.BlockSpec((t_blk, 8), lambda t, f: (jnp.maximum(t - 1, 0), 0)),
            pl.BlockSpec((t_blk, 8), lambda t, f: (jnp.maximum(t - 1, 0), 0)),
        ],
        out_shape=[
            jax.ShapeDtypeStruct((n, 8), jnp.int32),
            jax.ShapeDtypeStruct((n, 8), jnp.float32),
        ],
        scratch_shapes=[
            pltpu.VMEM((t_blk, d_model), jnp.float32),
            pltpu.VMEM((t_blk, d_model), jnp.bfloat16),
            pltpu.VMEM((2, t_blk, d_ff), jnp.float32),
            pltpu.VMEM((2, t_blk, d_ff), jnp.bfloat16),
            pltpu.VMEM((2, t_blk, 8), jnp.int32),
            pltpu.VMEM((2, t_blk, 8), jnp.float32),
        ],
        compiler_params=pltpu.CompilerParams(
            dimension_semantics=("arbitrary", "arbitrary")),
    )(x_flat, Wr1, Wr2, W_in.astype(jnp.bfloat16), ln_g.reshape(1, -1),
      ln_b.reshape(1, -1))


def _sc_gather(table, idx):
    v, d = table.shape
    b = idx.shape[0]
    info = plsc.get_sparse_core_info()
    nw = info.num_cores * info.num_subcores
    b_per_w = b // nw
    chunk = 256
    n_chunks = b_per_w // chunk
    mesh = plsc.VectorSubcoreMesh(core_axis_name="c", subcore_axis_name="s")

    @functools.partial(
        pl.kernel,
        mesh=mesh,
        out_type=jax.ShapeDtypeStruct((b, d), jnp.float32),
        scratch_types=[
            pltpu.VMEM((b_per_w,), jnp.int32),
            pltpu.VMEM((chunk, d), jnp.float32),
            pltpu.VMEM((chunk, d), jnp.float32),
            pltpu.SemaphoreType.DMA,
            pltpu.SemaphoreType.DMA,
        ],
    )
    def gather_kernel(table_hbm, idx_hbm, out_hbm, idx_v, rows0, rows1, sem0,
                      sem1):
        wid = lax.axis_index("s") * info.num_cores + lax.axis_index("c")
        base = wid * b_per_w
        pltpu.sync_copy(idx_hbm.at[pl.ds(base, b_per_w)], idx_v)
        bufs = (rows0, rows1)
        sems = (sem0, sem1)
        copies = [pltpu.async_copy(
            table_hbm.at[idx_v.at[pl.ds(0, chunk)]], bufs[0], sems[0])]
        for c in range(n_chunks):
            if c + 1 < n_chunks:
                copies.append(pltpu.async_copy(
                    table_hbm.at[idx_v.at[pl.ds((c + 1) * chunk, chunk)]],
                    bufs[(c + 1) % 2], sems[(c + 1) % 2]))
            copies[c].wait()
            pltpu.sync_copy(bufs[c % 2],
                            out_hbm.at[pl.ds(base + c * chunk, chunk)])

    return gather_kernel(table, idx)


def _phi_rho_body(nv_ref, act_ref, w1a_ref, wact_ref, b1_ref, g1_ref, bb1_ref,
                  w2_ref, b2_ref, g2_ref, bb2_ref, rg_ref, rb_ref,
                  rw1_ref, rb1_ref, rw2_ref, rb2_ref, out_ref):
    t1 = _dot_t(nv_ref[...], w1a_ref[...])
    t1 = t1 + act_ref[...] * wact_ref[...] + b1_ref[...]
    t1 = _gelu(_ln_rows(t1, g1_ref[...], bb1_ref[...]))
    t2 = _dot_t(t1, w2_ref[...]) + b2_ref[...]
    t2 = _ln_rows(t2, g2_ref[...], bb2_ref[...])
    rows = t2.shape[0]
    agg = jnp.sum(t2.reshape(rows // 8, 8, t2.shape[1]), axis=1)
    r = _ln_rows(agg, rg_ref[...], rb_ref[...])
    r = _gelu(_dot_t(r, rw1_ref[...]) + rb1_ref[...])
    out_ref[...] = _dot_t(r, rw2_ref[...]) + rb2_ref[...]


def _phi_rho(nv_sel, acts_col, phi_W1a, phi_wact, phi_b1, phi_ln1_g, phi_ln1_b,
             phi_W2, phi_b2, phi_ln2_g, phi_ln2_b, rho_ln_g, rho_ln_b,
             rho_W1, rho_b1, rho_W2, rho_b2, t_blk=512):
    rows, d_n = nv_sel.shape
    n = rows // 8
    d_h = phi_W2.shape[0]
    d_h2 = rho_W1.shape[0]
    d_model = rho_W2.shape[0]
    grid = (n // t_blk,)
    row_blk = t_blk * 8

    def full(shape):
        return pl.BlockSpec(shape, lambda t: tuple(0 for _ in shape))

    return pl.pallas_call(
        _phi_rho_body,
        grid=grid,
        in_specs=[
            pl.BlockSpec((row_blk, d_n), lambda t: (t, 0)),
            pl.BlockSpec((row_blk, 1), lambda t: (t, 0)),
            full((d_h, d_n)),
            full((1, d_h)),
            full((1, d_h)),
            full((1, d_h)),
            full((1, d_h)),
            full((d_h, d_h)),
            full((1, d_h)),
            full((1, d_h)),
            full((1, d_h)),
            full((1, d_h)),
            full((1, d_h)),
            full((d_h2, d_h)),
            full((1, d_h2)),
            full((d_model, d_h2)),
            full((1, d_model)),
        ],
        out_specs=pl.BlockSpec((t_blk, d_model), lambda t: (t, 0)),
        out_shape=jax.ShapeDtypeStruct((n, d_model), jnp.float32),
    )(nv_sel, acts_col, phi_W1a, phi_wact.reshape(1, -1),
      phi_b1.reshape(1, -1), phi_ln1_g.reshape(1, -1),
      phi_ln1_b.reshape(1, -1), phi_W2, phi_b2.reshape(1, -1),
      phi_ln2_g.reshape(1, -1), phi_ln2_b.reshape(1, -1),
      rho_ln_g.reshape(1, -1), rho_ln_b.reshape(1, -1),
      rho_W1, rho_b1.reshape(1, -1), rho_W2, rho_b2.reshape(1, -1))


def kernel(x, neuron_vecs, W_in, Wr1, Wr2, ln_r_g, ln_r_b, phi_W1, phi_b1,
           phi_ln1_g, phi_ln1_b, phi_W2, phi_b2, phi_ln2_g, phi_ln2_b,
           rho_ln_g, rho_ln_b, rho_W1, rho_b1, rho_W2, rho_b2, top_k):
    batch, seq, d_model = x.shape
    n = batch * seq
    x_flat = x.reshape(n, d_model)

    sel, acts = _router_topk(x_flat, Wr1, Wr2, W_in, ln_r_g, ln_r_b)

    nv_sel = _sc_gather(neuron_vecs, sel.reshape(n * 8))

    out = _phi_rho(
        nv_sel, acts.reshape(n * 8, 1),
        phi_W1[:, :neuron_vecs.shape[1]], phi_W1[:, neuron_vecs.shape[1]],
        phi_b1, phi_ln1_g, phi_ln1_b, phi_W2, phi_b2, phi_ln2_g, phi_ln2_b,
        rho_ln_g, rho_ln_b, rho_W1, rho_b1, rho_W2, rho_b2)
    return out.reshape(batch, seq, d_model)
